# Initial kernel scaffold; baseline (speedup 1.0000x reference)
import functools

import jax
import jax.numpy as jnp
from jax import lax
from jax.experimental import pallas as pl
from jax.experimental.pallas import tpu as pltpu
from jax.experimental.pallas import tpu_sc as plsc

N = 10000
E = 320000
R = 8
NB = 4
D = 128

NSLICES = 8
NSLICE = N // NSLICES
PASSES = NSLICES // 2
NTILES = 16
SCHUNK = 2000
ESHARE = E // NTILES
NSTAGE = ESHARE // SCHUNK
GCHUNK = 256
TRASH = R * NSLICE
ACC_ROWS = TRASH + 16
CAPCHUNKS = (ESHARE + 17 * 16 + GCHUNK - 1) // GCHUNK + 1
ROWS_OUT = TRASH // NTILES

_mesh = plsc.VectorSubcoreMesh(core_axis_name="c", subcore_axis_name="s")


def _zero_fill(buf, nvec):
    z = jnp.zeros((16,), jnp.float32)

    def body(i, _):
        buf[pl.ds(i * 16, 16)] = z
        return 0

    lax.fori_loop(0, nvec, body, 0)


@functools.partial(
    pl.kernel,
    out_type=jax.ShapeDtypeStruct((2, R * N), jnp.float32),
    mesh=_mesh,
    scratch_types=[
        pltpu.VMEM((2, SCHUNK), jnp.int32),
        pltpu.VMEM((SCHUNK,), jnp.int32),
        pltpu.VMEM((SCHUNK,), jnp.int32),
        pltpu.VMEM((SCHUNK,), jnp.float32),
        pltpu.VMEM((SCHUNK,), jnp.float32),
        pltpu.VMEM_SHARED((R * N,), jnp.float32),
    ],
)
def _count_kernel(ei_hbm, et_hbm, cnt_hbm, stage_sd, stage_t, seg_v, ones_v,
                  zbuf, cacc):
    c = lax.axis_index("c")
    s = lax.axis_index("s")
    _zero_fill(zbuf, SCHUNK // 16)
    one = jnp.full((16,), 1.0, jnp.float32)

    def fill_ones(i, _):
        ones_v[pl.ds(i * 16, 16)] = one
        return 0

    lax.fori_loop(0, SCHUNK // 16, fill_ones, 0)
    share = R * N // NTILES
    t0 = s * share
    pltpu.sync_copy(zbuf, cacc.at[pl.ds(t0, SCHUNK)])
    pltpu.sync_copy(zbuf, cacc.at[pl.ds(t0 + SCHUNK, SCHUNK)])
    pltpu.sync_copy(zbuf.at[pl.ds(0, share - 2 * SCHUNK)],
                    cacc.at[pl.ds(t0 + 2 * SCHUNK, share - 2 * SCHUNK)])
    plsc.subcore_barrier()

    nchunks = E // 2 // NTILES // SCHUNK

    def chunk(j, _):
        ebase = (c * NTILES + s) * (E // 2 // NTILES) + j * SCHUNK
        pltpu.sync_copy(ei_hbm.at[:, pl.ds(ebase, SCHUNK)], stage_sd)
        pltpu.sync_copy(et_hbm.at[pl.ds(ebase, SCHUNK)], stage_t)

        def vec(i, _):
            dst16 = stage_sd[1, pl.ds(i * 16, 16)]
            typ16 = stage_t[pl.ds(i * 16, 16)]
            seg_v[pl.ds(i * 16, 16)] = dst16 * R + typ16
            return 0

        lax.fori_loop(0, SCHUNK // 16, vec, 0)
        pltpu.sync_copy(ones_v, cacc.at[seg_v], add=True)
        return 0

    lax.fori_loop(0, nchunks, chunk, 0)
    plsc.subcore_barrier()
    pltpu.sync_copy(cacc.at[pl.ds(t0, share)], cnt_hbm.at[c, pl.ds(t0, share)])


@functools.partial(
    pl.kernel,
    out_type=jax.ShapeDtypeStruct((R * N, D), jnp.float32),
    mesh=_mesh,
    scratch_types=[
        pltpu.VMEM((2, SCHUNK), jnp.int32),
        pltpu.VMEM((SCHUNK,), jnp.int32),
        pltpu.VMEM((CAPCHUNKS, GCHUNK), jnp.int32),
        pltpu.VMEM((CAPCHUNKS, GCHUNK), jnp.int32),
        pltpu.VMEM((GCHUNK, D), jnp.float32),
        pltpu.VMEM((64, D), jnp.float32),
        pltpu.VMEM_SHARED((ACC_ROWS, D), jnp.float32),
        pltpu.SemaphoreType.DMA,
    ],
)
def _agg_kernel(x_hbm, ei_hbm, et_hbm, s_hbm, stage_sd, stage_t, comp_g,
                comp_s, rows_v, zrows, acc, sem):
    c = lax.axis_index("c")
    s = lax.axis_index("s")
    z16 = jnp.zeros((16,), jnp.float32)

    def zb(i, _):
        for k in range(D // 16):
            zrows[i, pl.ds(k * 16, 16)] = z16
        return 0

    lax.fori_loop(0, 64, zb, 0)
    iota16 = lax.iota(jnp.int32, 16)
    rows_per_tile = ACC_ROWS // NTILES

    for p in range(PASSES):
        base_node = (c * PASSES + p) * NSLICE

        t0 = s * rows_per_tile

        def zacc(i, _):
            pltpu.sync_copy(zrows, acc.at[pl.ds(t0 + i * 64, 64)])
            return 0

        lax.fori_loop(0, rows_per_tile // 64, zacc, 0)
        rem = rows_per_tile % 64
        pltpu.sync_copy(zrows.at[pl.ds(0, rem)],
                        acc.at[pl.ds(t0 + (rows_per_tile // 64) * 64, rem)])
        plsc.subcore_barrier()

        def stage(j, off):
            ebase = s * ESHARE + j * SCHUNK
            pltpu.sync_copy(ei_hbm.at[:, pl.ds(ebase, SCHUNK)], stage_sd)
            pltpu.sync_copy(et_hbm.at[pl.ds(ebase, SCHUNK)], stage_t)

            def vec(i, off):
                src16 = stage_sd[0, pl.ds(i * 16, 16)]
                dst16 = stage_sd[1, pl.ds(i * 16, 16)]
                typ16 = stage_t[pl.ds(i * 16, 16)]
                rel = dst16 - base_node
                mask = (rel >= 0) & (rel < NSLICE)
                sidx16 = typ16 * NSLICE + rel
                pos = plsc.cumsum(mask.astype(jnp.int32))
                tgt = off + pos - 1
                plsc.store_scatter(comp_g, [tgt // GCHUNK, tgt % GCHUNK],
                                   src16, mask=mask)
                plsc.store_scatter(comp_s, [tgt // GCHUNK, tgt % GCHUNK],
                                   sidx16, mask=mask)
                return off + pos[15]

            return lax.fori_loop(0, SCHUNK // 16, vec, off)

        m = lax.fori_loop(0, NSTAGE, stage, jnp.int32(0))

        trash_s = TRASH + iota16
        zero_g = jnp.zeros((16,), jnp.int32)

        def pad(i, _):
            tgt = m + i * 16 + iota16
            plsc.store_scatter(comp_g, [tgt // GCHUNK, tgt % GCHUNK], zero_g)
            plsc.store_scatter(comp_s, [tgt // GCHUNK, tgt % GCHUNK], trash_s)
            return 0

        lax.fori_loop(0, GCHUNK // 16 + 1, pad, 0)

        nchunks = (m + GCHUNK - 1) // GCHUNK

        def gchunk(k, _):
            pltpu.async_copy(x_hbm.at[comp_g.at[k]], rows_v, sem).wait()
            pltpu.sync_copy(rows_v, acc.at[comp_s.at[k]], add=True)
            return 0

        lax.fori_loop(0, nchunks, gchunk, 0)
        plsc.subcore_barrier()

        r = s // 2
        dst0 = r * N + base_node + (s % 2) * ROWS_OUT
        pltpu.sync_copy(acc.at[pl.ds(s * ROWS_OUT, ROWS_OUT)],
                        s_hbm.at[pl.ds(dst0, ROWS_OUT)])
        plsc.subcore_barrier()


BN = 400


def _combine_body(act, s_ref, h_ref, cnt_ref, basis_ref, comp_ref, root_ref,
                  bias_ref, o_ref):
    cnt = cnt_ref[0] + cnt_ref[1]
    invc = 1.0 / jnp.maximum(cnt, 1.0)
    acc = jnp.dot(h_ref[...], root_ref[...],
                  preferred_element_type=jnp.float32) + bias_ref[...]
    for r in range(R):
        w_r = comp_ref[r, 0] * basis_ref[0]
        for b in range(1, NB):
            w_r = w_r + comp_ref[r, b] * basis_ref[b]
        mean_r = s_ref[r] * invc[:, r][:, None]
        acc = acc + jnp.dot(mean_r, w_r, preferred_element_type=jnp.float32)
    if act == 0:
        o_ref[...] = jnp.maximum(acc, 0.0)
    else:
        mx = jnp.max(acc, axis=1, keepdims=True)
        e = jnp.exp(acc - mx)
        o_ref[...] = e / jnp.sum(e, axis=1, keepdims=True)


def _combine(act, s3, h, cnt3, basis, comp, root, bias):
    return pl.pallas_call(
        functools.partial(_combine_body, act),
        grid=(N // BN,),
        in_specs=[
            pl.BlockSpec((R, BN, D), lambda i: (0, i, 0)),
            pl.BlockSpec((BN, D), lambda i: (i, 0)),
            pl.BlockSpec((2, BN, R), lambda i: (0, i, 0)),
            pl.BlockSpec((NB, D, D), lambda i: (0, 0, 0)),
            pl.BlockSpec(memory_space=pltpu.SMEM),
            pl.BlockSpec((D, D), lambda i: (0, 0)),
            pl.BlockSpec((1, D), lambda i: (0, 0)),
        ],
        out_specs=pl.BlockSpec((BN, D), lambda i: (i, 0)),
        out_shape=jax.ShapeDtypeStruct((N, D), jnp.float32),
    )(s3, h, cnt3, basis, comp, root, bias.reshape(1, D))


def kernel(x, edge_index, edge_type, basis0, comp0, root0, bias0, basis1,
           comp1, root1, bias1, basis2, comp2, root2, bias2):
    cnt = _count_kernel(edge_index, edge_type)
    cnt3 = cnt.reshape(2, N, R)
    h = x
    layers = [(basis0, comp0, root0, bias0, 0),
              (basis1, comp1, root1, bias1, 0),
              (basis2, comp2, root2, bias2, 1)]
    for basis, comp, root, bias, act in layers:
        s_flat = _agg_kernel(h, edge_index, edge_type)
        s3 = s_flat.reshape(R, N, D)
        h = _combine(act, s3, h, cnt3, basis, comp, root, bias)
    return h


# speedup vs baseline: 5.9088x; 5.9088x over previous
import functools

import jax
import jax.numpy as jnp
from jax import lax
from jax.experimental import pallas as pl
from jax.experimental.pallas import tpu as pltpu
from jax.experimental.pallas import tpu_sc as plsc

N = 10000
E = 320000
R = 8
NB = 4
D = 128

NTILES = 16
NSC = 2
NW = NSC * NTILES
SCHUNK = 2000
GC = 128
NP = 10240

_mesh = plsc.VectorSubcoreMesh(core_axis_name="c", subcore_axis_name="s")


@functools.partial(
    pl.kernel,
    out_type=(jax.ShapeDtypeStruct((E,), jnp.float32),
              jax.ShapeDtypeStruct((E,), jnp.int32)),
    mesh=_mesh,
    scratch_types=[
        pltpu.VMEM((SCHUNK,), jnp.int32),
        pltpu.VMEM((SCHUNK,), jnp.int32),
        pltpu.VMEM((SCHUNK,), jnp.int32),
        pltpu.VMEM((SCHUNK,), jnp.int32),
        pltpu.VMEM((SCHUNK,), jnp.float32),
        pltpu.VMEM((SCHUNK,), jnp.float32),
        pltpu.VMEM((SCHUNK,), jnp.int32),
        pltpu.VMEM_SHARED((R * N,), jnp.float32),
    ],
)
def _prep_kernel(src_hbm, dst_hbm, et_hbm, w_hbm, gidx_hbm, stage_s, stage_d,
                 stage_t, seg_v, fbuf, wbuf, gbuf, cacc):
    c = lax.axis_index("c")
    s = lax.axis_index("s")
    z16 = jnp.zeros((16,), jnp.float32)
    one16 = jnp.full((16,), 1.0, jnp.float32)

    def fill_zero(i, _):
        fbuf[pl.ds(i * 16, 16)] = z16
        return 0

    lax.fori_loop(0, SCHUNK // 16, fill_zero, 0)
    share = R * N // NTILES
    t0 = s * share
    pltpu.sync_copy(fbuf, cacc.at[pl.ds(t0, SCHUNK)])
    pltpu.sync_copy(fbuf, cacc.at[pl.ds(t0 + SCHUNK, SCHUNK)])
    pltpu.sync_copy(fbuf.at[pl.ds(0, share - 2 * SCHUNK)],
                    cacc.at[pl.ds(t0 + 2 * SCHUNK, share - 2 * SCHUNK)])
    plsc.subcore_barrier()

    def fill_one(i, _):
        fbuf[pl.ds(i * 16, 16)] = one16
        return 0

    lax.fori_loop(0, SCHUNK // 16, fill_one, 0)

    def count_chunk(j, _):
        ebase = s * (E // NTILES) + j * SCHUNK
        pltpu.sync_copy(dst_hbm.at[pl.ds(ebase, SCHUNK)], stage_d)
        pltpu.sync_copy(et_hbm.at[pl.ds(ebase, SCHUNK)], stage_t)

        def vec(i, _):
            dst16 = stage_d[pl.ds(i * 16, 16)]
            typ16 = stage_t[pl.ds(i * 16, 16)]
            seg_v[pl.ds(i * 16, 16)] = dst16 * R + typ16
            return 0

        lax.fori_loop(0, SCHUNK // 16, vec, 0)
        pltpu.sync_copy(fbuf, cacc.at[seg_v], add=True)
        return 0

    lax.fori_loop(0, E // NTILES // SCHUNK, count_chunk, 0)
    plsc.subcore_barrier()

    wid = c * NTILES + s

    def emit_chunk(j, _):
        ebase = wid * (E // NW) + j * SCHUNK
        pltpu.sync_copy(src_hbm.at[pl.ds(ebase, SCHUNK)], stage_s)
        pltpu.sync_copy(dst_hbm.at[pl.ds(ebase, SCHUNK)], stage_d)
        pltpu.sync_copy(et_hbm.at[pl.ds(ebase, SCHUNK)], stage_t)

        def vseg(i, _):
            dst16 = stage_d[pl.ds(i * 16, 16)]
            typ16 = stage_t[pl.ds(i * 16, 16)]
            src16 = stage_s[pl.ds(i * 16, 16)]
            seg_v[pl.ds(i * 16, 16)] = dst16 * R + typ16
            gbuf[pl.ds(i * 16, 16)] = typ16 * N + src16
            return 0

        lax.fori_loop(0, SCHUNK // 16, vseg, 0)
        pltpu.sync_copy(cacc.at[seg_v], fbuf)

        def vw(i, _):
            cv = fbuf[pl.ds(i * 16, 16)]
            wbuf[pl.ds(i * 16, 16)] = 1.0 / jnp.maximum(cv, 1.0)
            return 0

        lax.fori_loop(0, SCHUNK // 16, vw, 0)
        pltpu.sync_copy(wbuf, w_hbm.at[pl.ds(ebase, SCHUNK)])
        pltpu.sync_copy(gbuf, gidx_hbm.at[pl.ds(ebase, SCHUNK)])
        return 0

    lax.fori_loop(0, E // NW // SCHUNK, emit_chunk, 0)


@functools.partial(
    pl.kernel,
    out_type=jax.ShapeDtypeStruct((2 * NP, D), jnp.float32),
    mesh=_mesh,
    scratch_types=[
        pltpu.VMEM((GC,), jnp.int32),
        pltpu.VMEM((GC,), jnp.int32),
        pltpu.VMEM((GC,), jnp.float32),
        pltpu.VMEM((16,), jnp.int32),
        pltpu.VMEM((16,), jnp.int32),
        pltpu.VMEM((16,), jnp.float32),
        pltpu.VMEM((GC, D), jnp.float32),
        pltpu.VMEM((16, D), jnp.float32),
        pltpu.VMEM((64, D), jnp.float32),
        pltpu.VMEM_SHARED((NP, D), jnp.float32),
        pltpu.SemaphoreType.DMA,
        pltpu.SemaphoreType.DMA,
    ],
)
def _agg_kernel(y_hbm, gidx_hbm, dst_hbm, w_hbm, out_hbm, gbuf, dbuf, wbuf,
                gbuf16, dbuf16, wbuf16, rows_v, rows16_v, zbuf, acc, sem,
                sem2):
    c = lax.axis_index("c")
    s = lax.axis_index("s")
    wid = c * NTILES + s
    eshare = E // NW
    nfull = eshare // GC
    z16 = jnp.zeros((16,), jnp.float32)

    def zb(i, _):
        for k in range(D // 16):
            zbuf[i, pl.ds(k * 16, 16)] = z16
        return 0

    lax.fori_loop(0, 64, zb, 0)

    t0 = s * (NP // NTILES)

    def zacc(i, _):
        pltpu.sync_copy(zbuf, acc.at[pl.ds(t0 + i * 64, 64)])
        return 0

    lax.fori_loop(0, NP // NTILES // 64, zacc, 0)
    plsc.subcore_barrier()

    def scale_rows(rv, wb, ngroups):
        def scale(g, _):
            w16 = wb[pl.ds(g * 16, 16)]
            for jj in range(16):
                wv = w16[jj]
                for kk in range(D // 16):
                    rv[g * 16 + jj, pl.ds(kk * 16, 16)] = (
                        rv[g * 16 + jj, pl.ds(kk * 16, 16)] * wv)
            return 0

        lax.fori_loop(0, ngroups, scale, 0)

    def chunk(k, _):
        ebase = wid * eshare + k * GC
        pltpu.async_copy(gidx_hbm.at[pl.ds(ebase, GC)], gbuf, sem2)
        pltpu.async_copy(dst_hbm.at[pl.ds(ebase, GC)], dbuf, sem2)
        cw = pltpu.async_copy(w_hbm.at[pl.ds(ebase, GC)], wbuf, sem2)
        cw.wait()
        cw.wait()
        cw.wait()
        pltpu.async_copy(y_hbm.at[gbuf], rows_v, sem).wait()
        scale_rows(rows_v, wbuf, GC // 16)
        pltpu.sync_copy(rows_v, acc.at[dbuf], add=True)
        return 0

    lax.fori_loop(0, nfull, chunk, 0)

    tbase = wid * eshare + nfull * GC
    pltpu.async_copy(gidx_hbm.at[pl.ds(tbase, 16)], gbuf16, sem2)
    pltpu.async_copy(dst_hbm.at[pl.ds(tbase, 16)], dbuf16, sem2)
    ct = pltpu.async_copy(w_hbm.at[pl.ds(tbase, 16)], wbuf16, sem2)
    ct.wait()
    ct.wait()
    ct.wait()
    pltpu.async_copy(y_hbm.at[gbuf16], rows16_v, sem).wait()
    scale_rows(rows16_v, wbuf16, 1)
    pltpu.sync_copy(rows16_v, acc.at[dbuf16], add=True)

    plsc.subcore_barrier()

    o0 = c * NP + s * (NP // NTILES)

    def ocp(i, _):
        pltpu.sync_copy(acc.at[pl.ds(t0 + i * 64, 64)], zbuf)
        pltpu.sync_copy(zbuf, out_hbm.at[pl.ds(o0 + i * 64, 64)])
        return 0

    lax.fori_loop(0, NP // NTILES // 64, ocp, 0)


BN = 400


def _y_body(h_ref, basis_ref, comp_ref, y_ref):
    h = h_ref[...]
    for r in range(R):
        w_r = comp_ref[r, 0] * basis_ref[0]
        for b in range(1, NB):
            w_r = w_r + comp_ref[r, b] * basis_ref[b]
        y_ref[r] = jnp.dot(h, w_r, preferred_element_type=jnp.float32, precision=lax.Precision.HIGHEST)


def _y_kernel(h, basis, comp):
    return pl.pallas_call(
        _y_body,
        grid=(N // BN,),
        in_specs=[
            pl.BlockSpec((BN, D), lambda i: (i, 0)),
            pl.BlockSpec((NB, D, D), lambda i: (0, 0, 0)),
            pl.BlockSpec(memory_space=pltpu.SMEM),
        ],
        out_specs=pl.BlockSpec((R, BN, D), lambda i: (0, i, 0)),
        out_shape=jax.ShapeDtypeStruct((R, N, D), jnp.float32),
    )(h, basis, comp)


def _finish_body(act, p0_ref, p1_ref, h_ref, root_ref, bias_ref, o_ref):
    acc = p0_ref[...] + p1_ref[...] + bias_ref[...] + jnp.dot(
        h_ref[...], root_ref[...], preferred_element_type=jnp.float32,
        precision=lax.Precision.HIGHEST)
    if act == 0:
        o_ref[...] = jnp.maximum(acc, 0.0)
    else:
        mx = jnp.max(acc, axis=1, keepdims=True)
        e = jnp.exp(acc - mx)
        o_ref[...] = e / jnp.sum(e, axis=1, keepdims=True)


def _finish(act, p0, p1, h, root, bias):
    return pl.pallas_call(
        functools.partial(_finish_body, act),
        grid=(N // BN,),
        in_specs=[
            pl.BlockSpec((BN, D), lambda i: (i, 0)),
            pl.BlockSpec((BN, D), lambda i: (i, 0)),
            pl.BlockSpec((BN, D), lambda i: (i, 0)),
            pl.BlockSpec((D, D), lambda i: (0, 0)),
            pl.BlockSpec((1, D), lambda i: (0, 0)),
        ],
        out_specs=pl.BlockSpec((BN, D), lambda i: (i, 0)),
        out_shape=jax.ShapeDtypeStruct((N, D), jnp.float32),
    )(p0, p1, h, root, bias.reshape(1, D))


def kernel(x, edge_index, edge_type, basis0, comp0, root0, bias0, basis1,
           comp1, root1, bias1, basis2, comp2, root2, bias2):
    src = edge_index[0]
    dst = edge_index[1]
    w, gidx = _prep_kernel(src, dst, edge_type)
    h = x
    layers = [(basis0, comp0, root0, bias0, 0),
              (basis1, comp1, root1, bias1, 0),
              (basis2, comp2, root2, bias2, 1)]
    for basis, comp, root, bias, act in layers:
        y = _y_kernel(h, basis, comp).reshape(R * N, D)
        parts = _agg_kernel(y, gidx, dst, w)
        h = _finish(act, parts[:N], parts[NP:NP + N], h, root, bias)
    return h
